# baseline (speedup 1.0000x reference)
import functools

import jax
import jax.numpy as jnp
from jax import lax
from jax.experimental import pallas as pl
from jax.experimental.pallas import tpu as pltpu
from jax.experimental.pallas import tpu_sc as plsc

_E = 64
_D = 4096
_L = 16
_NW = 32
_NCHUNK = 4


def _logits_body(x_ref, w_ref, out_ref):
    out_ref[...] = lax.dot_general(
        x_ref[...], w_ref[...],
        dimension_numbers=(((1,), (1,)), ((), ())),
        preferred_element_type=jnp.float32)


def _router_logits(x, w, blk, chunk, nchunk):
    t = x.shape[0]
    tc = t // nchunk
    off = chunk * (tc // blk)
    return pl.pallas_call(
        _logits_body,
        grid=(tc // blk,),
        in_specs=[
            pl.BlockSpec((blk, _D), lambda i: (off + i, 0)),
            pl.BlockSpec((_E, _D), lambda i: (0, 0)),
        ],
        out_specs=pl.BlockSpec((blk, _E), lambda i: (i, 0)),
        out_shape=jax.ShapeDtypeStruct((tc, _E), jnp.float32),
    )(x, w)


def _merge(a, b):
    m1a, i1a, m2a, i2a = a
    m1b, i1b, m2b, i2b = b
    ge = m1a >= m1b
    m1 = jnp.where(ge, m1a, m1b)
    i1 = jnp.where(ge, i1a, i1b)
    ge2a = m2a >= m1b
    ge2b = m1a >= m2b
    m2 = jnp.where(ge, jnp.where(ge2a, m2a, m1b), jnp.where(ge2b, m1a, m2b))
    i2 = jnp.where(ge, jnp.where(ge2a, i2a, i1b), jnp.where(ge2b, i1a, i2b))
    return m1, i1, m2, i2


def _make_router(t):
    tok_w = t // _NW
    mesh = plsc.VectorSubcoreMesh(core_axis_name="c", subcore_axis_name="s")

    @functools.partial(
        pl.kernel,
        mesh=mesh,
        out_type=[jax.ShapeDtypeStruct((t * 2,), jnp.float32),
                  jax.ShapeDtypeStruct((t * 2,), jnp.int32)],
        scratch_types=[pltpu.VMEM((tok_w * _E,), jnp.float32),
                       pltpu.VMEM((tok_w * 2,), jnp.float32),
                       pltpu.VMEM((tok_w * 2,), jnp.int32)],
        compiler_params=pltpu.CompilerParams(needs_layout_passes=False),
    )
    def route(logits_hbm, w_hbm, i_hbm, buf, wbuf, ibuf):
        wid = lax.axis_index("s") * 2 + lax.axis_index("c")
        base = wid * tok_w
        pltpu.sync_copy(logits_hbm.at[pl.ds(base * _E, tok_w * _E)], buf)
        lanes = lax.iota(jnp.int32, _L)

        def group(g, carry):
            flat = (g * _L + lanes) * _E
            neg = jnp.full((_L,), -3.0e38, jnp.float32)
            zero = jnp.zeros((_L,), jnp.int32)

            def expert(j, c):
                out = []
                for k in range(4):
                    m1, i1, m2, i2 = c[4 * k:4 * k + 4]
                    col = jnp.full((_L,), j + 16 * k, jnp.int32)
                    v = plsc.load_gather(buf, [flat + col])
                    gt1 = v > m1
                    gt2 = v > m2
                    nm2 = jnp.where(gt1, m1, jnp.where(gt2, v, m2))
                    ni2 = jnp.where(gt1, i1, jnp.where(gt2, col, i2))
                    nm1 = jnp.where(gt1, v, m1)
                    ni1 = jnp.where(gt1, col, i1)
                    out += [nm1, ni1, nm2, ni2]
                return tuple(out)

            init = (neg, zero, neg, zero) * 4
            c = lax.fori_loop(0, 16, expert, init, unroll=4)
            ab = _merge(c[0:4], c[4:8])
            cd = _merge(c[8:12], c[12:16])
            m1, i1, m2, i2 = _merge(ab, cd)
            e2 = jnp.exp(m2 - m1)
            w1 = 1.0 / (1.0 + e2)
            w2 = 1.0 - w1
            row2 = (g * _L + lanes) * 2
            plsc.store_scatter(wbuf, [row2], w1)
            plsc.store_scatter(wbuf, [row2 + 1], w2)
            plsc.store_scatter(ibuf, [row2], i1)
            plsc.store_scatter(ibuf, [row2 + 1], i2)
            return carry

        lax.fori_loop(0, tok_w // _L, group, 0)
        pltpu.sync_copy(wbuf, w_hbm.at[pl.ds(base * 2, tok_w * 2)])
        pltpu.sync_copy(ibuf, i_hbm.at[pl.ds(base * 2, tok_w * 2)])

    return route


def kernel(hidden_states, router_weight):
    b, s, d = hidden_states.shape
    t = b * s
    x = hidden_states.reshape(t, d)
    tc = t // _NCHUNK
    route = _make_router(tc)
    ws, idxs, lgs = [], [], []
    for c in range(_NCHUNK):
        lg = _router_logits(x, router_weight, 256, c, _NCHUNK)
        w, i = route(lg.reshape(tc * _E))
        ws.append(w.reshape(tc, 2))
        idxs.append(i.reshape(tc, 2))
        lgs.append(lg)
    w = jnp.concatenate(ws).reshape(b, s, 2)
    idx = jnp.concatenate(idxs).reshape(b, s, 2)
    logits = jnp.concatenate(lgs).reshape(b, s, _E)
    return (w, idx, logits)


# speedup vs baseline: 1.0055x; 1.0055x over previous
import functools

import jax
import jax.numpy as jnp
from jax import lax
from jax.experimental import pallas as pl
from jax.experimental.pallas import tpu as pltpu
from jax.experimental.pallas import tpu_sc as plsc

_E = 64
_D = 4096
_L = 16
_NW = 32
_NCHUNK = 8


def _logits_body(x_ref, w_ref, out_ref):
    out_ref[...] = lax.dot_general(
        x_ref[...], w_ref[...],
        dimension_numbers=(((1,), (1,)), ((), ())),
        preferred_element_type=jnp.float32)


def _router_logits(x, w, blk, chunk, nchunk):
    t = x.shape[0]
    tc = t // nchunk
    off = chunk * (tc // blk)
    return pl.pallas_call(
        _logits_body,
        grid=(tc // blk,),
        in_specs=[
            pl.BlockSpec((blk, _D), lambda i: (off + i, 0)),
            pl.BlockSpec((_E, _D), lambda i: (0, 0)),
        ],
        out_specs=pl.BlockSpec((blk, _E), lambda i: (i, 0)),
        out_shape=jax.ShapeDtypeStruct((tc, _E), jnp.float32),
    )(x, w)


def _merge(a, b):
    m1a, i1a, m2a, i2a = a
    m1b, i1b, m2b, i2b = b
    ge = m1a >= m1b
    m1 = jnp.where(ge, m1a, m1b)
    i1 = jnp.where(ge, i1a, i1b)
    ge2a = m2a >= m1b
    ge2b = m1a >= m2b
    m2 = jnp.where(ge, jnp.where(ge2a, m2a, m1b), jnp.where(ge2b, m1a, m2b))
    i2 = jnp.where(ge, jnp.where(ge2a, i2a, i1b), jnp.where(ge2b, i1a, i2b))
    return m1, i1, m2, i2


def _make_router(t):
    tok_w = t // _NW
    mesh = plsc.VectorSubcoreMesh(core_axis_name="c", subcore_axis_name="s")

    @functools.partial(
        pl.kernel,
        mesh=mesh,
        out_type=[jax.ShapeDtypeStruct((t * 2,), jnp.float32),
                  jax.ShapeDtypeStruct((t * 2,), jnp.int32)],
        scratch_types=[pltpu.VMEM((tok_w * _E,), jnp.float32),
                       pltpu.VMEM((tok_w * 2,), jnp.float32),
                       pltpu.VMEM((tok_w * 2,), jnp.int32)],
        compiler_params=pltpu.CompilerParams(needs_layout_passes=False),
    )
    def route(logits_hbm, w_hbm, i_hbm, buf, wbuf, ibuf):
        wid = lax.axis_index("s") * 2 + lax.axis_index("c")
        base = wid * tok_w
        pltpu.sync_copy(logits_hbm.at[pl.ds(base * _E, tok_w * _E)], buf)
        lanes = lax.iota(jnp.int32, _L)

        def group(g, carry):
            flat = (g * _L + lanes) * _E
            neg = jnp.full((_L,), -3.0e38, jnp.float32)
            zero = jnp.zeros((_L,), jnp.int32)

            def expert(j, c):
                out = []
                for k in range(4):
                    m1, i1, m2, i2 = c[4 * k:4 * k + 4]
                    col = jnp.full((_L,), j + 16 * k, jnp.int32)
                    v = plsc.load_gather(buf, [flat + col])
                    gt1 = v > m1
                    gt2 = v > m2
                    nm2 = jnp.where(gt1, m1, jnp.where(gt2, v, m2))
                    ni2 = jnp.where(gt1, i1, jnp.where(gt2, col, i2))
                    nm1 = jnp.where(gt1, v, m1)
                    ni1 = jnp.where(gt1, col, i1)
                    out += [nm1, ni1, nm2, ni2]
                return tuple(out)

            init = (neg, zero, neg, zero) * 4
            c = lax.fori_loop(0, 16, expert, init, unroll=4)
            ab = _merge(c[0:4], c[4:8])
            cd = _merge(c[8:12], c[12:16])
            m1, i1, m2, i2 = _merge(ab, cd)
            e2 = jnp.exp(m2 - m1)
            w1 = 1.0 / (1.0 + e2)
            w2 = 1.0 - w1
            row2 = (g * _L + lanes) * 2
            plsc.store_scatter(wbuf, [row2], w1)
            plsc.store_scatter(wbuf, [row2 + 1], w2)
            plsc.store_scatter(ibuf, [row2], i1)
            plsc.store_scatter(ibuf, [row2 + 1], i2)
            return carry

        lax.fori_loop(0, tok_w // _L, group, 0)
        pltpu.sync_copy(wbuf, w_hbm.at[pl.ds(base * 2, tok_w * 2)])
        pltpu.sync_copy(ibuf, i_hbm.at[pl.ds(base * 2, tok_w * 2)])

    return route


def kernel(hidden_states, router_weight):
    b, s, d = hidden_states.shape
    t = b * s
    x = hidden_states.reshape(t, d)
    tc = t // _NCHUNK
    route = _make_router(tc)
    ws, idxs, lgs = [], [], []
    for c in range(_NCHUNK):
        lg = _router_logits(x, router_weight, 512, c, _NCHUNK)
        w, i = route(lg.reshape(tc * _E))
        ws.append(w.reshape(tc, 2))
        idxs.append(i.reshape(tc, 2))
        lgs.append(lg)
    w = jnp.concatenate(ws).reshape(b, s, 2)
    idx = jnp.concatenate(idxs).reshape(b, s, 2)
    logits = jnp.concatenate(lgs).reshape(b, s, _E)
    return (w, idx, logits)


# speedup vs baseline: 1.1148x; 1.1086x over previous
import functools

import jax
import jax.numpy as jnp
from jax import lax
from jax.experimental import pallas as pl
from jax.experimental.pallas import tpu as pltpu
from jax.experimental.pallas import tpu_sc as plsc

_E = 64
_D = 4096
_L = 16
_NW = 32
_NCHUNK = 2


def _logits_body(x_ref, w_ref, out_ref):
    out_ref[...] = lax.dot_general(
        x_ref[...], w_ref[...],
        dimension_numbers=(((1,), (1,)), ((), ())),
        preferred_element_type=jnp.float32)


def _router_logits(x, w, blk, chunk, nchunk):
    t = x.shape[0]
    tc = t // nchunk
    off = chunk * (tc // blk)
    return pl.pallas_call(
        _logits_body,
        grid=(tc // blk,),
        in_specs=[
            pl.BlockSpec((blk, _D), lambda i: (off + i, 0)),
            pl.BlockSpec((_E, _D), lambda i: (0, 0)),
        ],
        out_specs=pl.BlockSpec((blk, _E), lambda i: (i, 0)),
        out_shape=jax.ShapeDtypeStruct((tc, _E), jnp.float32),
    )(x, w)


def _merge(a, b):
    m1a, i1a, m2a, i2a = a
    m1b, i1b, m2b, i2b = b
    ge = m1a >= m1b
    m1 = jnp.where(ge, m1a, m1b)
    i1 = jnp.where(ge, i1a, i1b)
    ge2a = m2a >= m1b
    ge2b = m1a >= m2b
    m2 = jnp.where(ge, jnp.where(ge2a, m2a, m1b), jnp.where(ge2b, m1a, m2b))
    i2 = jnp.where(ge, jnp.where(ge2a, i2a, i1b), jnp.where(ge2b, i1a, i2b))
    return m1, i1, m2, i2


def _make_router(t):
    tok_w = t // _NW
    mesh = plsc.VectorSubcoreMesh(core_axis_name="c", subcore_axis_name="s")

    @functools.partial(
        pl.kernel,
        mesh=mesh,
        out_type=[jax.ShapeDtypeStruct((t * 2,), jnp.float32),
                  jax.ShapeDtypeStruct((t * 2,), jnp.int32)],
        scratch_types=[pltpu.VMEM((tok_w * _E,), jnp.float32),
                       pltpu.VMEM((tok_w * 2,), jnp.float32),
                       pltpu.VMEM((tok_w * 2,), jnp.int32)],
        compiler_params=pltpu.CompilerParams(needs_layout_passes=False),
    )
    def route(logits_hbm, w_hbm, i_hbm, buf, wbuf, ibuf):
        wid = lax.axis_index("s") * 2 + lax.axis_index("c")
        base = wid * tok_w
        pltpu.sync_copy(logits_hbm.at[pl.ds(base * _E, tok_w * _E)], buf)
        lanes = lax.iota(jnp.int32, _L)

        def group(g, carry):
            flat = (g * _L + lanes) * _E
            neg = jnp.full((_L,), -3.0e38, jnp.float32)
            zero = jnp.zeros((_L,), jnp.int32)

            def expert(j, c):
                out = []
                for k in range(4):
                    m1, i1, m2, i2 = c[4 * k:4 * k + 4]
                    col = jnp.full((_L,), j + 16 * k, jnp.int32)
                    v = plsc.load_gather(buf, [flat + col])
                    gt1 = v > m1
                    gt2 = v > m2
                    nm2 = jnp.where(gt1, m1, jnp.where(gt2, v, m2))
                    ni2 = jnp.where(gt1, i1, jnp.where(gt2, col, i2))
                    nm1 = jnp.where(gt1, v, m1)
                    ni1 = jnp.where(gt1, col, i1)
                    out += [nm1, ni1, nm2, ni2]
                return tuple(out)

            init = (neg, zero, neg, zero) * 4
            c = lax.fori_loop(0, 16, expert, init, unroll=4)
            ab = _merge(c[0:4], c[4:8])
            cd = _merge(c[8:12], c[12:16])
            m1, i1, m2, i2 = _merge(ab, cd)
            e2 = jnp.exp(m2 - m1)
            w1 = 1.0 / (1.0 + e2)
            w2 = 1.0 - w1
            row2 = (g * _L + lanes) * 2
            plsc.store_scatter(wbuf, [row2], w1)
            plsc.store_scatter(wbuf, [row2 + 1], w2)
            plsc.store_scatter(ibuf, [row2], i1)
            plsc.store_scatter(ibuf, [row2 + 1], i2)
            return carry

        lax.fori_loop(0, tok_w // _L, group, 0)
        pltpu.sync_copy(wbuf, w_hbm.at[pl.ds(base * 2, tok_w * 2)])
        pltpu.sync_copy(ibuf, i_hbm.at[pl.ds(base * 2, tok_w * 2)])

    return route


def kernel(hidden_states, router_weight):
    b, s, d = hidden_states.shape
    t = b * s
    x = hidden_states.reshape(t, d)
    tc = t // _NCHUNK
    route = _make_router(tc)
    ws, idxs, lgs = [], [], []
    for c in range(_NCHUNK):
        lg = _router_logits(x, router_weight, 512, c, _NCHUNK)
        w, i = route(lg.reshape(tc * _E))
        ws.append(w.reshape(tc, 2))
        idxs.append(i.reshape(tc, 2))
        lgs.append(lg)
    w = jnp.concatenate(ws).reshape(b, s, 2)
    idx = jnp.concatenate(idxs).reshape(b, s, 2)
    logits = jnp.concatenate(lgs).reshape(b, s, _E)
    return (w, idx, logits)


# speedup vs baseline: 1.1526x; 1.0340x over previous
import functools

import jax
import jax.numpy as jnp
from jax import lax
from jax.experimental import pallas as pl
from jax.experimental.pallas import tpu as pltpu
from jax.experimental.pallas import tpu_sc as plsc

_E = 64
_D = 4096
_L = 16
_NW = 32
_NCHUNK = 2


def _logits_body(x_ref, w_ref, out_ref, wide_ref):
    lg = lax.dot_general(
        x_ref[...], w_ref[...],
        dimension_numbers=(((1,), (1,)), ((), ())),
        preferred_element_type=jnp.float32)
    out_ref[...] = lg
    wide_ref[...] = jnp.concatenate([lg, lg], axis=1)


def _router_logits(x, w, blk, chunk, nchunk):
    t = x.shape[0]
    tc = t // nchunk
    off = chunk * (tc // blk)
    return pl.pallas_call(
        _logits_body,
        grid=(tc // blk,),
        in_specs=[
            pl.BlockSpec((blk, _D), lambda i: (off + i, 0)),
            pl.BlockSpec((_E, _D), lambda i: (0, 0)),
        ],
        out_specs=[
            pl.BlockSpec((blk, _E), lambda i: (i, 0)),
            pl.BlockSpec((blk, 2 * _E), lambda i: (i, 0)),
        ],
        out_shape=[
            jax.ShapeDtypeStruct((tc, _E), jnp.float32),
            jax.ShapeDtypeStruct((tc, 2 * _E), jnp.float32),
        ],
    )(x, w)


def _merge(a, b):
    m1a, i1a, m2a, i2a = a
    m1b, i1b, m2b, i2b = b
    ge = m1a >= m1b
    m1 = jnp.where(ge, m1a, m1b)
    i1 = jnp.where(ge, i1a, i1b)
    ge2a = m2a >= m1b
    ge2b = m1a >= m2b
    m2 = jnp.where(ge, jnp.where(ge2a, m2a, m1b), jnp.where(ge2b, m1a, m2b))
    i2 = jnp.where(ge, jnp.where(ge2a, i2a, i1b), jnp.where(ge2b, i1a, i2b))
    return m1, i1, m2, i2


def _make_router(t):
    tok_w = t // _NW
    mesh = plsc.VectorSubcoreMesh(core_axis_name="c", subcore_axis_name="s")

    @functools.partial(
        pl.kernel,
        mesh=mesh,
        out_type=[jax.ShapeDtypeStruct((t * 2,), jnp.float32),
                  jax.ShapeDtypeStruct((t * 2,), jnp.int32)],
        scratch_types=[pltpu.VMEM((tok_w * 2 * _E,), jnp.float32),
                       pltpu.VMEM((tok_w * 2,), jnp.float32),
                       pltpu.VMEM((tok_w * 2,), jnp.int32)],
        compiler_params=pltpu.CompilerParams(needs_layout_passes=False),
    )
    def route(logits_hbm, w_hbm, i_hbm, buf, wbuf, ibuf):
        wid = lax.axis_index("s") * 2 + lax.axis_index("c")
        base = wid * tok_w
        pltpu.sync_copy(
            logits_hbm.at[pl.ds(base * 2 * _E, tok_w * 2 * _E)], buf)
        lanes = lax.iota(jnp.int32, _L)

        def group(g, carry):
            flat = (g * _L + lanes) * (2 * _E)
            neg = jnp.full((_L,), -3.0e38, jnp.float32)
            zero = jnp.zeros((_L,), jnp.int32)

            def expert(j, c):
                out = []
                for k in range(4):
                    m1, i1, m2, i2 = c[4 * k:4 * k + 4]
                    col = jnp.full((_L,), j + 16 * k, jnp.int32)
                    v = plsc.load_gather(buf, [flat + col])
                    gt1 = v > m1
                    gt2 = v > m2
                    nm2 = jnp.where(gt1, m1, jnp.where(gt2, v, m2))
                    ni2 = jnp.where(gt1, i1, jnp.where(gt2, col, i2))
                    nm1 = jnp.where(gt1, v, m1)
                    ni1 = jnp.where(gt1, col, i1)
                    out += [nm1, ni1, nm2, ni2]
                return tuple(out)

            init = (neg, zero, neg, zero) * 4
            c = lax.fori_loop(0, 16, expert, init, unroll=4)
            ab = _merge(c[0:4], c[4:8])
            cd = _merge(c[8:12], c[12:16])
            m1, i1, m2, i2 = _merge(ab, cd)
            e2 = jnp.exp(m2 - m1)
            w1 = 1.0 / (1.0 + e2)
            w2 = 1.0 - w1
            row2 = (g * _L + lanes) * 2
            plsc.store_scatter(wbuf, [row2], w1)
            plsc.store_scatter(wbuf, [row2 + 1], w2)
            plsc.store_scatter(ibuf, [row2], i1)
            plsc.store_scatter(ibuf, [row2 + 1], i2)
            return carry

        lax.fori_loop(0, tok_w // _L, group, 0)
        pltpu.sync_copy(wbuf, w_hbm.at[pl.ds(base * 2, tok_w * 2)])
        pltpu.sync_copy(ibuf, i_hbm.at[pl.ds(base * 2, tok_w * 2)])

    return route


def kernel(hidden_states, router_weight):
    b, s, d = hidden_states.shape
    t = b * s
    x = hidden_states.reshape(t, d)
    tc = t // _NCHUNK
    route = _make_router(tc)
    ws, idxs, lgs = [], [], []
    for c in range(_NCHUNK):
        lg, lgp = _router_logits(x, router_weight, 512, c, _NCHUNK)
        w, i = route(lgp.reshape(tc * 2 * _E))
        ws.append(w.reshape(tc, 2))
        idxs.append(i.reshape(tc, 2))
        lgs.append(lg)
    w = jnp.concatenate(ws).reshape(b, s, 2)
    idx = jnp.concatenate(idxs).reshape(b, s, 2)
    logits = jnp.concatenate(lgs).reshape(b, s, _E)
    return (w, idx, logits)


# speedup vs baseline: 1.1579x; 1.0045x over previous
import functools

import jax
import jax.numpy as jnp
from jax import lax
from jax.experimental import pallas as pl
from jax.experimental.pallas import tpu as pltpu
from jax.experimental.pallas import tpu_sc as plsc

_E = 64
_D = 4096
_L = 16
_NW = 32
_NCHUNK = 2


def _logits_body(x_ref, w_ref, out_ref, wide_ref):
    lg = lax.dot_general(
        x_ref[...], w_ref[...],
        dimension_numbers=(((1,), (1,)), ((), ())),
        preferred_element_type=jnp.float32)
    out_ref[...] = lg
    wide_ref[...] = jnp.concatenate([lg, lg], axis=1)


def _router_logits(x, w, blk, tok_off, tc):
    off = tok_off // blk
    return pl.pallas_call(
        _logits_body,
        grid=(tc // blk,),
        in_specs=[
            pl.BlockSpec((blk, _D), lambda i: (off + i, 0)),
            pl.BlockSpec((_E, _D), lambda i: (0, 0)),
        ],
        out_specs=[
            pl.BlockSpec((blk, _E), lambda i: (i, 0)),
            pl.BlockSpec((blk, 2 * _E), lambda i: (i, 0)),
        ],
        out_shape=[
            jax.ShapeDtypeStruct((tc, _E), jnp.float32),
            jax.ShapeDtypeStruct((tc, 2 * _E), jnp.float32),
        ],
    )(x, w)


def _merge(a, b):
    m1a, i1a, m2a, i2a = a
    m1b, i1b, m2b, i2b = b
    ge = m1a >= m1b
    m1 = jnp.where(ge, m1a, m1b)
    i1 = jnp.where(ge, i1a, i1b)
    ge2a = m2a >= m1b
    ge2b = m1a >= m2b
    m2 = jnp.where(ge, jnp.where(ge2a, m2a, m1b), jnp.where(ge2b, m1a, m2b))
    i2 = jnp.where(ge, jnp.where(ge2a, i2a, i1b), jnp.where(ge2b, i1a, i2b))
    return m1, i1, m2, i2


def _make_router(t):
    tok_w = t // _NW
    mesh = plsc.VectorSubcoreMesh(core_axis_name="c", subcore_axis_name="s")

    half = tok_w // 2
    row_w = 2 * _E

    @functools.partial(
        pl.kernel,
        mesh=mesh,
        out_type=[jax.ShapeDtypeStruct((t * 2,), jnp.float32),
                  jax.ShapeDtypeStruct((t * 2,), jnp.int32)],
        scratch_types=[pltpu.VMEM((half * row_w,), jnp.float32),
                       pltpu.VMEM((half * row_w,), jnp.float32),
                       pltpu.VMEM((tok_w * 2,), jnp.float32),
                       pltpu.VMEM((tok_w * 2,), jnp.int32),
                       pltpu.SemaphoreType.DMA,
                       pltpu.SemaphoreType.DMA],
        compiler_params=pltpu.CompilerParams(needs_layout_passes=False),
    )
    def route(logits_hbm, w_hbm, i_hbm, buf0, buf1, wbuf, ibuf, sem0, sem1):
        wid = lax.axis_index("s") * 2 + lax.axis_index("c")
        base = wid * tok_w
        cp0 = pltpu.async_copy(
            logits_hbm.at[pl.ds(base * row_w, half * row_w)], buf0, sem0)
        cp1 = pltpu.async_copy(
            logits_hbm.at[pl.ds((base + half) * row_w, half * row_w)],
            buf1, sem1)
        lanes = lax.iota(jnp.int32, _L)

        def process(buf, tok_off):
            def group(g, carry):
                row = g * _L + lanes
                neg = jnp.full((_L,), -3.0e38, jnp.float32)
                zero = jnp.zeros((_L,), jnp.int32)

                def expert(j, c):
                    out = []
                    for k in range(4):
                        m1, i1, m2, i2 = c[4 * k:4 * k + 4]
                        col = jnp.full((_L,), j + 16 * k, jnp.int32)
                        v = plsc.load_gather(buf, [row * row_w + col])
                        gt1 = v > m1
                        gt2 = v > m2
                        nm2 = jnp.where(gt1, m1, jnp.where(gt2, v, m2))
                        ni2 = jnp.where(gt1, i1, jnp.where(gt2, col, i2))
                        nm1 = jnp.where(gt1, v, m1)
                        ni1 = jnp.where(gt1, col, i1)
                        out += [nm1, ni1, nm2, ni2]
                    return tuple(out)

                init = (neg, zero, neg, zero) * 4
                c = lax.fori_loop(0, 16, expert, init, unroll=4)
                ab = _merge(c[0:4], c[4:8])
                cd = _merge(c[8:12], c[12:16])
                m1, i1, m2, i2 = _merge(ab, cd)
                e2 = jnp.exp(m2 - m1)
                w1 = 1.0 / (1.0 + e2)
                w2 = 1.0 - w1
                row2 = (tok_off + g * _L + lanes) * 2
                plsc.store_scatter(wbuf, [row2], w1)
                plsc.store_scatter(wbuf, [row2 + 1], w2)
                plsc.store_scatter(ibuf, [row2], i1)
                plsc.store_scatter(ibuf, [row2 + 1], i2)
                return carry

            lax.fori_loop(0, half // _L, group, 0)

        cp0.wait()
        process(buf0, 0)
        cp1.wait()
        process(buf1, half)
        pltpu.sync_copy(wbuf, w_hbm.at[pl.ds(base * 2, tok_w * 2)])
        pltpu.sync_copy(ibuf, i_hbm.at[pl.ds(base * 2, tok_w * 2)])

    return route


def kernel(hidden_states, router_weight):
    b, s, d = hidden_states.shape
    t = b * s
    x = hidden_states.reshape(t, d)
    sizes = [3 * t // 4, t // 4]
    offs = [0, 3 * t // 4]
    ws, idxs, lgs = [], [], []
    for tok_off, tc in zip(offs, sizes):
        lg, lgp = _router_logits(x, router_weight, 512, tok_off, tc)
        w, i = _make_router(tc)(lgp.reshape(tc * 2 * _E))
        ws.append(w.reshape(tc, 2))
        idxs.append(i.reshape(tc, 2))
        lgs.append(lg)
    w = jnp.concatenate(ws).reshape(b, s, 2)
    idx = jnp.concatenate(idxs).reshape(b, s, 2)
    logits = jnp.concatenate(lgs).reshape(b, s, _E)
    return (w, idx, logits)


# speedup vs baseline: 1.1706x; 1.0110x over previous
import functools

import jax
import jax.numpy as jnp
from jax import lax
from jax.experimental import pallas as pl
from jax.experimental.pallas import tpu as pltpu
from jax.experimental.pallas import tpu_sc as plsc

_E = 64
_D = 4096
_L = 16
_NW = 32
_NCHUNK = 2


def _logits_body(x_ref, w_ref, out_ref, wide_ref):
    lg = lax.dot_general(
        x_ref[...], w_ref[...],
        dimension_numbers=(((1,), (1,)), ((), ())),
        preferred_element_type=jnp.float32)
    out_ref[...] = lg
    wide_ref[...] = jnp.concatenate([lg, lg], axis=1)


def _router_logits(x, w, blk, tok_off, tc):
    off = tok_off // blk
    return pl.pallas_call(
        _logits_body,
        grid=(tc // blk,),
        in_specs=[
            pl.BlockSpec((blk, _D), lambda i: (off + i, 0)),
            pl.BlockSpec((_E, _D), lambda i: (0, 0)),
        ],
        out_specs=[
            pl.BlockSpec((blk, _E), lambda i: (i, 0)),
            pl.BlockSpec((blk, 2 * _E), lambda i: (i, 0)),
        ],
        out_shape=[
            jax.ShapeDtypeStruct((tc, _E), jnp.float32),
            jax.ShapeDtypeStruct((tc, 2 * _E), jnp.float32),
        ],
    )(x, w)


def _merge(a, b):
    m1a, i1a, m2a, i2a = a
    m1b, i1b, m2b, i2b = b
    ge = m1a >= m1b
    m1 = jnp.where(ge, m1a, m1b)
    i1 = jnp.where(ge, i1a, i1b)
    ge2a = m2a >= m1b
    ge2b = m1a >= m2b
    m2 = jnp.where(ge, jnp.where(ge2a, m2a, m1b), jnp.where(ge2b, m1a, m2b))
    i2 = jnp.where(ge, jnp.where(ge2a, i2a, i1b), jnp.where(ge2b, i1a, i2b))
    return m1, i1, m2, i2


def _make_router(t):
    tok_w = t // _NW
    mesh = plsc.VectorSubcoreMesh(core_axis_name="c", subcore_axis_name="s")

    half = tok_w // 2
    row_w = 2 * _E

    @functools.partial(
        pl.kernel,
        mesh=mesh,
        out_type=[jax.ShapeDtypeStruct((t * 2,), jnp.float32),
                  jax.ShapeDtypeStruct((t * 2,), jnp.int32)],
        scratch_types=[pltpu.VMEM((half * row_w,), jnp.float32),
                       pltpu.VMEM((half * row_w,), jnp.float32),
                       pltpu.VMEM((tok_w * 2,), jnp.float32),
                       pltpu.VMEM((tok_w * 2,), jnp.int32),
                       pltpu.SemaphoreType.DMA,
                       pltpu.SemaphoreType.DMA],
        compiler_params=pltpu.CompilerParams(needs_layout_passes=False),
    )
    def route(logits_hbm, w_hbm, i_hbm, buf0, buf1, wbuf, ibuf, sem0, sem1):
        wid = lax.axis_index("s") * 2 + lax.axis_index("c")
        base = wid * tok_w
        cp0 = pltpu.async_copy(
            logits_hbm.at[pl.ds(base * row_w, half * row_w)], buf0, sem0)
        cp1 = pltpu.async_copy(
            logits_hbm.at[pl.ds((base + half) * row_w, half * row_w)],
            buf1, sem1)
        lanes = lax.iota(jnp.int32, _L)

        def process(buf, tok_off):
            def group(g, carry):
                row = g * _L + lanes
                rowbase = row * row_w
                neg = jnp.full((_L,), -3.0e38, jnp.float32)
                zero = jnp.zeros((_L,), jnp.int32)

                def expert(j, c):
                    sh = lanes + j
                    sh = jnp.where(sh >= _L, sh - _L, sh)
                    out = []
                    for k in range(4):
                        m1, i1, m2, i2 = c[4 * k:4 * k + 4]
                        col = sh + 16 * k
                        v = plsc.load_gather(buf, [rowbase + col])
                        gt1 = v > m1
                        gt2 = v > m2
                        nm2 = jnp.where(gt1, m1, jnp.where(gt2, v, m2))
                        ni2 = jnp.where(gt1, i1, jnp.where(gt2, col, i2))
                        nm1 = jnp.where(gt1, v, m1)
                        ni1 = jnp.where(gt1, col, i1)
                        out += [nm1, ni1, nm2, ni2]
                    return tuple(out)

                init = (neg, zero, neg, zero) * 4
                c = lax.fori_loop(0, 16, expert, init, unroll=4)
                ab = _merge(c[0:4], c[4:8])
                cd = _merge(c[8:12], c[12:16])
                m1, i1, m2, i2 = _merge(ab, cd)
                e2 = jnp.exp(m2 - m1)
                w1 = 1.0 / (1.0 + e2)
                w2 = 1.0 - w1
                row2 = (tok_off + g * _L + lanes) * 2
                plsc.store_scatter(wbuf, [row2], w1)
                plsc.store_scatter(wbuf, [row2 + 1], w2)
                plsc.store_scatter(ibuf, [row2], i1)
                plsc.store_scatter(ibuf, [row2 + 1], i2)
                return carry

            lax.fori_loop(0, half // _L, group, 0)

        cp0.wait()
        process(buf0, 0)
        cp1.wait()
        process(buf1, half)
        pltpu.sync_copy(wbuf, w_hbm.at[pl.ds(base * 2, tok_w * 2)])
        pltpu.sync_copy(ibuf, i_hbm.at[pl.ds(base * 2, tok_w * 2)])

    return route


def kernel(hidden_states, router_weight):
    b, s, d = hidden_states.shape
    t = b * s
    x = hidden_states.reshape(t, d)
    sizes = [3 * t // 4, t // 4]
    offs = [0, 3 * t // 4]
    ws, idxs, lgs = [], [], []
    for tok_off, tc in zip(offs, sizes):
        lg, lgp = _router_logits(x, router_weight, 512, tok_off, tc)
        w, i = _make_router(tc)(lgp.reshape(tc * 2 * _E))
        ws.append(w.reshape(tc, 2))
        idxs.append(i.reshape(tc, 2))
        lgs.append(lg)
    w = jnp.concatenate(ws).reshape(b, s, 2)
    idx = jnp.concatenate(idxs).reshape(b, s, 2)
    logits = jnp.concatenate(lgs).reshape(b, s, _E)
    return (w, idx, logits)
